# baseline (speedup 1.0000x reference)
import functools

import jax
import jax.numpy as jnp
from jax import lax
from jax.experimental import pallas as pl
from jax.experimental.pallas import tpu as pltpu
from jax.experimental.pallas import tpu_sc as plsc

B, N, G, K = 16, 8192, 256, 32
_NC, _NS, _L = 2, 16, 16
_NW = _NC * _NS


_W = 256
_T = N // _W


def _fps_body(xp, yp, zp, cx_ref, cy_ref, cz_ref, d_ref):
    d_ref[...] = jnp.full((B, N), 1e10, jnp.float32)
    jj = lax.broadcasted_iota(jnp.int32, (B, G), 1)
    li = lax.broadcasted_iota(jnp.int32, (B, _W), 1)

    def body(t, carry):
        px, py, pz, cx, cy, cz = carry
        sl = jj == t
        cx = jnp.where(sl, px, cx)
        cy = jnp.where(sl, py, cy)
        cz = jnp.where(sl, pz, cz)
        Mv = Iv = Xv = Yv = Zv = None
        for i in range(_T):
            s = pl.ds(i * _W, _W)
            xs = xp[:, s]
            ys = yp[:, s]
            zs = zp[:, s]
            dx = xs - px
            dy = ys - py
            dz = zs - pz
            dn = dx * dx + dy * dy + dz * dz
            dt = jnp.minimum(d_ref[:, s], dn)
            d_ref[:, s] = dt
            if i == 0:
                Mv, Xv, Yv, Zv = dt, xs, ys, zs
                Iv = li
            else:
                upd = dt > Mv
                Mv = jnp.where(upd, dt, Mv)
                Iv = jnp.where(upd, i * _W + li, Iv)
                Xv = jnp.where(upd, xs, Xv)
                Yv = jnp.where(upd, ys, Yv)
                Zv = jnp.where(upd, zs, Zv)
        mx = jnp.max(Mv, axis=1, keepdims=True)
        cand = jnp.where(Mv == mx, Iv, N)
        g = jnp.min(cand, axis=1, keepdims=True)
        sel = cand == g
        px = jnp.sum(jnp.where(sel, Xv, 0.0), axis=1, keepdims=True)
        py = jnp.sum(jnp.where(sel, Yv, 0.0), axis=1, keepdims=True)
        pz = jnp.sum(jnp.where(sel, Zv, 0.0), axis=1, keepdims=True)
        return px, py, pz, cx, cy, cz

    czero = jnp.zeros((B, G), jnp.float32)
    p0 = (xp[:, pl.ds(0, 1)], yp[:, pl.ds(0, 1)], zp[:, pl.ds(0, 1)])
    _, _, _, cx, cy, cz = lax.fori_loop(
        0, G, body, (*p0, czero, czero, czero))
    cx_ref[...] = cx
    cy_ref[...] = cy
    cz_ref[...] = cz


def _fps_call(x, y, z):
    out = jax.ShapeDtypeStruct((B, G), jnp.float32)
    return pl.pallas_call(
        _fps_body,
        out_shape=(out, out, out),
        scratch_shapes=[pltpu.VMEM((B, N), jnp.float32)],
    )(x, y, z)


CH = 16
NCH = N // CH


def _chunk_body(xp, yp, zp, cxp, cyp, czp, ids_ref, c_ref):
    x = xp[0]
    y = yp[0]
    z = zp[0]
    cx = cxp[0][:, :, None]
    cy = cyp[0][:, :, None]
    cz = czp[0][:, :, None]
    dx = cx - x[None]
    dy = cy - y[None]
    dz = cz - z[None]
    c_ref[...] = jnp.transpose(
        jnp.min(dx * dx + dy * dy + dz * dz, axis=1))
    si = lax.broadcasted_iota(jnp.int32, (NCH, G), 0)
    ki = lax.broadcasted_iota(jnp.int32, (K, G), 0)

    def body(k, carry):
        ids, prev = carry
        c = jnp.where(si == prev, jnp.inf, c_ref[...])
        c_ref[...] = c
        m = jnp.min(c, axis=0, keepdims=True)
        cid = jnp.min(jnp.where(c == m, si, NCH), axis=0, keepdims=True)
        ids = jnp.where(ki == k, cid, ids)
        return ids, cid

    ids_ref[0] = lax.fori_loop(
        0, K, body,
        (jnp.zeros((K, G), jnp.int32), jnp.full((1, G), -1, jnp.int32)))[0]


def _chunk_call(x, y, z, cx, cy, cz):
    vec = pl.BlockSpec((1, CH, NCH), lambda b: (b, 0, 0))
    cvec = pl.BlockSpec((1, G, 1), lambda b: (b, 0, 0))
    return pl.pallas_call(
        _chunk_body,
        grid=(B,),
        in_specs=[vec, vec, vec, cvec, cvec, cvec],
        out_specs=pl.BlockSpec((1, K, G), lambda b: (b, 0, 0)),
        out_shape=jax.ShapeDtypeStruct((B, K, G), jnp.int32),
        scratch_shapes=[pltpu.VMEM((NCH, G), jnp.float32)],
    )(x.reshape(B, CH, NCH), y.reshape(B, CH, NCH), z.reshape(B, CH, NCH),
      cx, cy, cz)


NCAND = K * CH


def _cand_body(xp, yp, zp, cid, cxh, cyh, czh, dv,
               xv, yv, zv, cidv, cxv, cyv, czv, odv):
    c = lax.axis_index("c")
    s = lax.axis_index("s")
    w = s * _NC + c
    b = w // _NC
    h = w % _NC
    gh = G // _NC
    pltpu.sync_copy(xp.at[b], xv)
    pltpu.sync_copy(yp.at[b], yv)
    pltpu.sync_copy(zp.at[b], zv)
    pltpu.sync_copy(cid.at[b, pl.ds(h * gh * K, gh * K)], cidv)
    pltpu.sync_copy(cxh.at[b, pl.ds(h * gh, gh)], cxv)
    pltpu.sync_copy(cyh.at[b, pl.ds(h * gh, gh)], cyv)
    pltpu.sync_copy(czh.at[b, pl.ds(h * gh, gh)], czv)

    def row(g, _):
        gsplat = jnp.full((_L,), g, jnp.int32)
        cgx = plsc.load_gather(cxv, [gsplat])
        cgy = plsc.load_gather(cyv, [gsplat])
        cgz = plsc.load_gather(czv, [gsplat])
        for j2 in range(K // _L):
            idvec = cidv[pl.ds(g * K + j2 * _L, _L)]
            pg = None
            for o in range(CH + 1):
                cur = None
                if o < CH:
                    gvec = idvec + NCH * o
                    cur = (o,
                           plsc.load_gather(xv, [gvec]),
                           plsc.load_gather(yv, [gvec]),
                           plsc.load_gather(zv, [gvec]))
                if pg is not None:
                    po, gx, gy, gz = pg
                    ddx = cgx - gx
                    ddy = cgy - gy
                    ddz = cgz - gz
                    odv[pl.ds(g * NCAND + po * K + j2 * _L, _L)] = (
                        ddx * ddx + ddy * ddy + ddz * ddz)
                pg = cur
        return 0

    lax.fori_loop(0, G // _NC, row, 0)
    pltpu.sync_copy(odv, dv.at[b, pl.ds(h * gh * NCAND, gh * NCAND)])


def _cand_call(x, y, z, cids_flat, cx, cy, cz):
    gh = G // _NC
    mesh = plsc.VectorSubcoreMesh(core_axis_name="c", subcore_axis_name="s")
    kfn = functools.partial(
        pl.kernel,
        mesh=mesh,
        compiler_params=pltpu.CompilerParams(needs_layout_passes=False),
        out_type=jax.ShapeDtypeStruct((B, G * NCAND), jnp.float32),
        scratch_types=[
            pltpu.VMEM((N,), jnp.float32),
            pltpu.VMEM((N,), jnp.float32),
            pltpu.VMEM((N,), jnp.float32),
            pltpu.VMEM((gh * K,), jnp.int32),
            pltpu.VMEM((gh,), jnp.float32),
            pltpu.VMEM((gh,), jnp.float32),
            pltpu.VMEM((gh,), jnp.float32),
            pltpu.VMEM((gh * NCAND,), jnp.float32),
        ],
    )(_cand_body)
    return kfn(x, y, z, cids_flat, cx, cy, cz)


def _knn_body(dvp, cidp, knn_ref, d_ref):
    d_ref[...] = jnp.transpose(dvp[0])
    cid = cidp[0]
    gi = jnp.concatenate([cid + NCH * o for o in range(CH)], axis=0)
    ki = lax.broadcasted_iota(jnp.int32, (K, G), 0)

    def body(k, carry):
        knn, prev = carry
        d = jnp.where(gi == prev, jnp.inf, d_ref[...])
        d_ref[...] = d
        m = jnp.min(d, axis=0, keepdims=True)
        gidx = jnp.min(jnp.where(d == m, gi, N), axis=0, keepdims=True)
        knn = jnp.where(ki == k, gidx, knn)
        return knn, gidx

    knn_ref[0] = lax.fori_loop(
        0, K, body,
        (jnp.zeros((K, G), jnp.int32), jnp.full((1, G), -1, jnp.int32)))[0]


def _knn_call(dv, cids):
    cand = pl.BlockSpec((1, G, NCAND), lambda b: (b, 0, 0))
    cidspec = pl.BlockSpec((1, K, G), lambda b: (b, 0, 0))
    return pl.pallas_call(
        _knn_body,
        grid=(B,),
        in_specs=[cand, cidspec],
        out_specs=pl.BlockSpec((1, K, G), lambda b: (b, 0, 0)),
        out_shape=jax.ShapeDtypeStruct((B, K, G), jnp.int32),
        scratch_shapes=[pltpu.VMEM((NCAND, G), jnp.float32)],
    )(dv, cids)


def _gather_body(xp, yp, zp, knn, cxh, cyh, czh, nx, ny, nz,
                 xv, yv, zv, idxv, cxv, cyv, czv, ox, oy, oz):
    c = lax.axis_index("c")
    s = lax.axis_index("s")
    w = s * _NC + c
    b = w // _NC
    h = w % _NC
    gh = G // _NC
    base = h * gh * K
    pltpu.sync_copy(xp.at[b], xv)
    pltpu.sync_copy(yp.at[b], yv)
    pltpu.sync_copy(zp.at[b], zv)
    pltpu.sync_copy(knn.at[b, pl.ds(base, gh * K)], idxv)
    pltpu.sync_copy(cxh.at[b, pl.ds(h * gh, gh)], cxv)
    pltpu.sync_copy(cyh.at[b, pl.ds(h * gh, gh)], cyv)
    pltpu.sync_copy(czh.at[b, pl.ds(h * gh, gh)], czv)

    def row(g, _):
        gsplat = jnp.full((_L,), g, jnp.int32)
        cgx = plsc.load_gather(cxv, [gsplat])
        cgy = plsc.load_gather(cyv, [gsplat])
        cgz = plsc.load_gather(czv, [gsplat])
        for j in range(K // _L):
            o = g * K + j * _L
            idx = idxv[pl.ds(o, _L)]
            ox[pl.ds(o, _L)] = plsc.load_gather(xv, [idx]) - cgx
            oy[pl.ds(o, _L)] = plsc.load_gather(yv, [idx]) - cgy
            oz[pl.ds(o, _L)] = plsc.load_gather(zv, [idx]) - cgz
        return 0

    lax.fori_loop(0, gh, row, 0)
    pltpu.sync_copy(ox, nx.at[b, pl.ds(base, gh * K)])
    pltpu.sync_copy(oy, ny.at[b, pl.ds(base, gh * K)])
    pltpu.sync_copy(oz, nz.at[b, pl.ds(base, gh * K)])


def _gather_call(x, y, z, knn_flat, cx, cy, cz):
    gh = G // _NC
    mesh = plsc.VectorSubcoreMesh(core_axis_name="c", subcore_axis_name="s")
    kfn = functools.partial(
        pl.kernel,
        mesh=mesh,
        compiler_params=pltpu.CompilerParams(needs_layout_passes=False),
        out_type=(jax.ShapeDtypeStruct((B, G * K), jnp.float32),) * 3,
        scratch_types=[
            pltpu.VMEM((N,), jnp.float32),
            pltpu.VMEM((N,), jnp.float32),
            pltpu.VMEM((N,), jnp.float32),
            pltpu.VMEM((gh * K,), jnp.int32),
            pltpu.VMEM((gh,), jnp.float32),
            pltpu.VMEM((gh,), jnp.float32),
            pltpu.VMEM((gh,), jnp.float32),
            pltpu.VMEM((gh * K,), jnp.float32),
            pltpu.VMEM((gh * K,), jnp.float32),
            pltpu.VMEM((gh * K,), jnp.float32),
        ],
    )(_gather_body)
    return kfn(x, y, z, knn_flat, cx, cy, cz)


def kernel(xyz):
    xyzt = jnp.transpose(xyz, (2, 0, 1))
    x, y, z = xyzt[0], xyzt[1], xyzt[2]
    cx, cy, cz = _fps_call(x, y, z)
    center = jnp.stack([cx, cy, cz], axis=-1)
    cids = _chunk_call(x, y, z, cx[..., None], cy[..., None], cz[..., None])
    cids_rows = jnp.transpose(cids, (0, 2, 1)).reshape(B, G * K)
    dv = _cand_call(x, y, z, cids_rows, cx, cy, cz)
    knn = _knn_call(dv.reshape(B, G, NCAND), cids)
    knn_rows = jnp.transpose(knn, (0, 2, 1)).reshape(B, G * K)
    nx, ny, nz = _gather_call(x, y, z, knn_rows, cx, cy, cz)
    neighborhood = jnp.stack([nx, ny, nz], axis=-1).reshape(B, G, K, 3)
    return neighborhood, center


# speedup vs baseline: 1.0561x; 1.0561x over previous
import functools

import jax
import jax.numpy as jnp
from jax import lax
from jax.experimental import pallas as pl
from jax.experimental.pallas import tpu as pltpu
from jax.experimental.pallas import tpu_sc as plsc

B, N, G, K = 16, 8192, 256, 32
_NC, _NS, _L = 2, 16, 16
_NW = _NC * _NS


_W = 256
_T = N // _W


def _fps_body(xp, yp, zp, cx_ref, cy_ref, cz_ref, d_ref):
    d_ref[...] = jnp.full((B, N), 1e10, jnp.float32)
    jj = lax.broadcasted_iota(jnp.int32, (B, G), 1)
    li = lax.broadcasted_iota(jnp.int32, (B, _W), 1)

    def body(t, carry):
        px, py, pz, cx, cy, cz = carry
        sl = jj == t
        cx = jnp.where(sl, px, cx)
        cy = jnp.where(sl, py, cy)
        cz = jnp.where(sl, pz, cz)
        Mv = Iv = Xv = Yv = Zv = None
        for i in range(_T):
            s = pl.ds(i * _W, _W)
            xs = xp[:, s]
            ys = yp[:, s]
            zs = zp[:, s]
            dx = xs - px
            dy = ys - py
            dz = zs - pz
            dn = dx * dx + dy * dy + dz * dz
            dt = jnp.minimum(d_ref[:, s], dn)
            d_ref[:, s] = dt
            if i == 0:
                Mv, Xv, Yv, Zv = dt, xs, ys, zs
                Iv = li
            else:
                upd = dt > Mv
                Mv = jnp.where(upd, dt, Mv)
                Iv = jnp.where(upd, i * _W + li, Iv)
                Xv = jnp.where(upd, xs, Xv)
                Yv = jnp.where(upd, ys, Yv)
                Zv = jnp.where(upd, zs, Zv)
        mx = jnp.max(Mv, axis=1, keepdims=True)
        cand = jnp.where(Mv == mx, Iv, N)
        g = jnp.min(cand, axis=1, keepdims=True)
        sel = cand == g
        px = jnp.sum(jnp.where(sel, Xv, 0.0), axis=1, keepdims=True)
        py = jnp.sum(jnp.where(sel, Yv, 0.0), axis=1, keepdims=True)
        pz = jnp.sum(jnp.where(sel, Zv, 0.0), axis=1, keepdims=True)
        return px, py, pz, cx, cy, cz

    czero = jnp.zeros((B, G), jnp.float32)
    p0 = (xp[:, pl.ds(0, 1)], yp[:, pl.ds(0, 1)], zp[:, pl.ds(0, 1)])
    _, _, _, cx, cy, cz = lax.fori_loop(
        0, G, body, (*p0, czero, czero, czero))
    cx_ref[...] = cx
    cy_ref[...] = cy
    cz_ref[...] = cz


def _fps_call(x, y, z):
    out = jax.ShapeDtypeStruct((B, G), jnp.float32)
    return pl.pallas_call(
        _fps_body,
        out_shape=(out, out, out),
        scratch_shapes=[pltpu.VMEM((B, N), jnp.float32)],
    )(x, y, z)


CH = 16
NCH = N // CH


def _chunk_body(xp, yp, zp, cxp, cyp, czp, ids_ref, c_ref):
    x = xp[0]
    y = yp[0]
    z = zp[0]
    cx = cxp[0][:, :, None]
    cy = cyp[0][:, :, None]
    cz = czp[0][:, :, None]
    dx = cx - x[None]
    dy = cy - y[None]
    dz = cz - z[None]
    c_ref[...] = jnp.transpose(
        jnp.min(dx * dx + dy * dy + dz * dz, axis=1))
    si = lax.broadcasted_iota(jnp.int32, (NCH, G), 0)
    ki = lax.broadcasted_iota(jnp.int32, (K, G), 0)

    def body(k, carry):
        ids, prev = carry
        c = jnp.where(si == prev, jnp.inf, c_ref[...])
        c_ref[...] = c
        m = jnp.min(c, axis=0, keepdims=True)
        cid = jnp.min(jnp.where(c == m, si, NCH), axis=0, keepdims=True)
        ids = jnp.where(ki == k, cid, ids)
        return ids, cid

    ids_ref[0] = lax.fori_loop(
        0, K, body,
        (jnp.zeros((K, G), jnp.int32), jnp.full((1, G), -1, jnp.int32)))[0]


def _chunk_call(x, y, z, cx, cy, cz):
    vec = pl.BlockSpec((1, CH, NCH), lambda b: (b, 0, 0))
    cvec = pl.BlockSpec((1, G, 1), lambda b: (b, 0, 0))
    nb = x.shape[0]
    return pl.pallas_call(
        _chunk_body,
        grid=(nb,),
        in_specs=[vec, vec, vec, cvec, cvec, cvec],
        out_specs=pl.BlockSpec((1, K, G), lambda b: (b, 0, 0)),
        out_shape=jax.ShapeDtypeStruct((nb, K, G), jnp.int32),
        scratch_shapes=[pltpu.VMEM((NCH, G), jnp.float32)],
    )(x.reshape(nb, CH, NCH), y.reshape(nb, CH, NCH), z.reshape(nb, CH, NCH),
      cx, cy, cz)


NCAND = K * CH


def _make_cand_body(wpb, gh):
  def _cand_body(xp, yp, zp, cid, cxh, cyh, czh, dv,
                 xv, yv, zv, cidv, cxv, cyv, czv, odv):
    c = lax.axis_index("c")
    s = lax.axis_index("s")
    w = s * _NC + c
    b = w // wpb
    h = w % wpb
    pltpu.sync_copy(xp.at[b], xv)
    pltpu.sync_copy(yp.at[b], yv)
    pltpu.sync_copy(zp.at[b], zv)
    pltpu.sync_copy(cid.at[b, pl.ds(h * gh * K, gh * K)], cidv)
    pltpu.sync_copy(cxh.at[b, pl.ds(h * gh, gh)], cxv)
    pltpu.sync_copy(cyh.at[b, pl.ds(h * gh, gh)], cyv)
    pltpu.sync_copy(czh.at[b, pl.ds(h * gh, gh)], czv)

    def row(g, _):
        gsplat = jnp.full((_L,), g, jnp.int32)
        cgx = plsc.load_gather(cxv, [gsplat])
        cgy = plsc.load_gather(cyv, [gsplat])
        cgz = plsc.load_gather(czv, [gsplat])
        for j2 in range(K // _L):
            idvec = cidv[pl.ds(g * K + j2 * _L, _L)]
            pg = None
            for o in range(CH + 1):
                cur = None
                if o < CH:
                    gvec = idvec + NCH * o
                    cur = (o,
                           plsc.load_gather(xv, [gvec]),
                           plsc.load_gather(yv, [gvec]),
                           plsc.load_gather(zv, [gvec]))
                if pg is not None:
                    po, gx, gy, gz = pg
                    ddx = cgx - gx
                    ddy = cgy - gy
                    ddz = cgz - gz
                    odv[pl.ds(g * NCAND + po * K + j2 * _L, _L)] = (
                        ddx * ddx + ddy * ddy + ddz * ddz)
                pg = cur
        return 0

    lax.fori_loop(0, gh, row, 0)
    pltpu.sync_copy(odv, dv.at[b, pl.ds(h * gh * NCAND, gh * NCAND)])
  return _cand_body


def _cand_call(x, y, z, cids_flat, cx, cy, cz):
    nb = x.shape[0]
    wpb = _NW // nb
    gh = G // wpb
    mesh = plsc.VectorSubcoreMesh(core_axis_name="c", subcore_axis_name="s")
    kfn = functools.partial(
        pl.kernel,
        mesh=mesh,
        compiler_params=pltpu.CompilerParams(needs_layout_passes=False),
        out_type=jax.ShapeDtypeStruct((nb, G * NCAND), jnp.float32),
        scratch_types=[
            pltpu.VMEM((N,), jnp.float32),
            pltpu.VMEM((N,), jnp.float32),
            pltpu.VMEM((N,), jnp.float32),
            pltpu.VMEM((gh * K,), jnp.int32),
            pltpu.VMEM((gh,), jnp.float32),
            pltpu.VMEM((gh,), jnp.float32),
            pltpu.VMEM((gh,), jnp.float32),
            pltpu.VMEM((gh * NCAND,), jnp.float32),
        ],
    )(_make_cand_body(wpb, gh))
    return kfn(x, y, z, cids_flat, cx, cy, cz)


def _knn_body(dvp, cidp, knn_ref, d_ref):
    d_ref[...] = jnp.transpose(dvp[0])
    cid = cidp[0]
    gi = jnp.concatenate([cid + NCH * o for o in range(CH)], axis=0)
    ki = lax.broadcasted_iota(jnp.int32, (K, G), 0)

    def body(k, carry):
        knn, prev = carry
        d = jnp.where(gi == prev, jnp.inf, d_ref[...])
        d_ref[...] = d
        m = jnp.min(d, axis=0, keepdims=True)
        gidx = jnp.min(jnp.where(d == m, gi, N), axis=0, keepdims=True)
        knn = jnp.where(ki == k, gidx, knn)
        return knn, gidx

    knn_ref[0] = lax.fori_loop(
        0, K, body,
        (jnp.zeros((K, G), jnp.int32), jnp.full((1, G), -1, jnp.int32)))[0]


def _knn_call(dv, cids):
    cand = pl.BlockSpec((1, G, NCAND), lambda b: (b, 0, 0))
    cidspec = pl.BlockSpec((1, K, G), lambda b: (b, 0, 0))
    nb = dv.shape[0]
    return pl.pallas_call(
        _knn_body,
        grid=(nb,),
        in_specs=[cand, cidspec],
        out_specs=pl.BlockSpec((1, K, G), lambda b: (b, 0, 0)),
        out_shape=jax.ShapeDtypeStruct((nb, K, G), jnp.int32),
        scratch_shapes=[pltpu.VMEM((NCAND, G), jnp.float32)],
    )(dv, cids)


def _make_gather_body(wpb, gh):
  def _gather_body(xp, yp, zp, knn, cxh, cyh, czh, nx, ny, nz,
                   xv, yv, zv, idxv, cxv, cyv, czv, ox, oy, oz):
    c = lax.axis_index("c")
    s = lax.axis_index("s")
    w = s * _NC + c
    b = w // wpb
    h = w % wpb
    base = h * gh * K
    pltpu.sync_copy(xp.at[b], xv)
    pltpu.sync_copy(yp.at[b], yv)
    pltpu.sync_copy(zp.at[b], zv)
    pltpu.sync_copy(knn.at[b, pl.ds(base, gh * K)], idxv)
    pltpu.sync_copy(cxh.at[b, pl.ds(h * gh, gh)], cxv)
    pltpu.sync_copy(cyh.at[b, pl.ds(h * gh, gh)], cyv)
    pltpu.sync_copy(czh.at[b, pl.ds(h * gh, gh)], czv)

    def row(g, _):
        gsplat = jnp.full((_L,), g, jnp.int32)
        cgx = plsc.load_gather(cxv, [gsplat])
        cgy = plsc.load_gather(cyv, [gsplat])
        cgz = plsc.load_gather(czv, [gsplat])
        for j in range(K // _L):
            o = g * K + j * _L
            idx = idxv[pl.ds(o, _L)]
            ox[pl.ds(o, _L)] = plsc.load_gather(xv, [idx]) - cgx
            oy[pl.ds(o, _L)] = plsc.load_gather(yv, [idx]) - cgy
            oz[pl.ds(o, _L)] = plsc.load_gather(zv, [idx]) - cgz
        return 0

    lax.fori_loop(0, gh, row, 0)
    pltpu.sync_copy(ox, nx.at[b, pl.ds(base, gh * K)])
    pltpu.sync_copy(oy, ny.at[b, pl.ds(base, gh * K)])
    pltpu.sync_copy(oz, nz.at[b, pl.ds(base, gh * K)])
  return _gather_body


def _gather_call(x, y, z, knn_flat, cx, cy, cz):
    nb = x.shape[0]
    wpb = _NW // nb
    gh = G // wpb
    mesh = plsc.VectorSubcoreMesh(core_axis_name="c", subcore_axis_name="s")
    kfn = functools.partial(
        pl.kernel,
        mesh=mesh,
        compiler_params=pltpu.CompilerParams(needs_layout_passes=False),
        out_type=(jax.ShapeDtypeStruct((nb, G * K), jnp.float32),) * 3,
        scratch_types=[
            pltpu.VMEM((N,), jnp.float32),
            pltpu.VMEM((N,), jnp.float32),
            pltpu.VMEM((N,), jnp.float32),
            pltpu.VMEM((gh * K,), jnp.int32),
            pltpu.VMEM((gh,), jnp.float32),
            pltpu.VMEM((gh,), jnp.float32),
            pltpu.VMEM((gh,), jnp.float32),
            pltpu.VMEM((gh * K,), jnp.float32),
            pltpu.VMEM((gh * K,), jnp.float32),
            pltpu.VMEM((gh * K,), jnp.float32),
        ],
    )(_make_gather_body(wpb, gh))
    return kfn(x, y, z, knn_flat, cx, cy, cz)


def kernel(xyz):
    xyzt = jnp.transpose(xyz, (2, 0, 1))
    x, y, z = xyzt[0], xyzt[1], xyzt[2]
    cx, cy, cz = _fps_call(x, y, z)
    center = jnp.stack([cx, cy, cz], axis=-1)
    hb = B // 2
    neigh = []
    for lo in (0, hb):
        sl = slice(lo, lo + hb)
        xs, ys, zs = x[sl], y[sl], z[sl]
        cxs, cys, czs = cx[sl], cy[sl], cz[sl]
        cids = _chunk_call(xs, ys, zs,
                           cxs[..., None], cys[..., None], czs[..., None])
        cids_rows = jnp.transpose(cids, (0, 2, 1)).reshape(hb, G * K)
        dv = _cand_call(xs, ys, zs, cids_rows, cxs, cys, czs)
        knn = _knn_call(dv.reshape(hb, G, NCAND), cids)
        knn_rows = jnp.transpose(knn, (0, 2, 1)).reshape(hb, G * K)
        nx, ny, nz = _gather_call(xs, ys, zs, knn_rows, cxs, cys, czs)
        neigh.append(jnp.stack([nx, ny, nz], axis=-1).reshape(hb, G, K, 3))
    return jnp.concatenate(neigh, axis=0), center
